# baseline (speedup 1.0000x reference)
import jax
import jax.numpy as jnp
from jax.experimental import pallas as pl
from jax.experimental.pallas import tpu as pltpu

D_MODEL = 512
S = 64
D2 = D_MODEL // 2


def _add_pos_kernel(x_ref, row_ref, col_ref, out_ref):
    ci = pl.program_id(0)

    @pl.when(ci == 0)
    def _():
        colT = col_ref[...].T
        pos = jnp.broadcast_to(colT[:, None, :], (D2, S, S))
        out_ref[0] = x_ref[0] + pos

    @pl.when(ci == 1)
    def _():
        rowT = row_ref[...].T
        pos = jnp.broadcast_to(rowT[:, :, None], (D2, S, S))
        out_ref[0] = x_ref[0] + pos


def kernel(x, row_embed, col_embed):
    B = x.shape[0]
    grid = (2, B)
    return pl.pallas_call(
        _add_pos_kernel,
        grid=grid,
        in_specs=[
            pl.BlockSpec((1, D2, S, S), lambda ci, b: (b, ci, 0, 0)),
            pl.BlockSpec((S, D2), lambda ci, b: (0, 0)),
            pl.BlockSpec((S, D2), lambda ci, b: (0, 0)),
        ],
        out_specs=pl.BlockSpec((1, D2, S, S), lambda ci, b: (b, ci, 0, 0)),
        out_shape=jax.ShapeDtypeStruct(x.shape, x.dtype),
        compiler_params=pltpu.CompilerParams(
            dimension_semantics=("parallel", "parallel"),
        ),
    )(x, row_embed, col_embed)


# speedup vs baseline: 1.7609x; 1.7609x over previous
import jax
import jax.numpy as jnp
from jax.experimental import pallas as pl
from jax.experimental.pallas import tpu as pltpu

D_MODEL = 512
S = 64
SS = S * S
D2 = D_MODEL // 2


def _build_pos_kernel(row_ref, col_ref, pos_ref):
    colT = col_ref[...].T
    rowT = row_ref[...].T
    pos_col = jnp.broadcast_to(colT[:, None, :], (D2, S, S)).reshape(D2, SS)
    pos_row = jnp.broadcast_to(rowT[:, :, None], (D2, S, S)).reshape(D2, SS)
    pos_ref[...] = jnp.concatenate([pos_col, pos_row], axis=0)


def _add_kernel(x_ref, pos_ref, out_ref):
    out_ref[0] = x_ref[0] + pos_ref[...]


def kernel(x, row_embed, col_embed):
    B = x.shape[0]
    pos = pl.pallas_call(
        _build_pos_kernel,
        out_shape=jax.ShapeDtypeStruct((D_MODEL, SS), x.dtype),
    )(row_embed, col_embed)

    xf = x.reshape(B, D_MODEL, SS)
    out = pl.pallas_call(
        _add_kernel,
        grid=(2, B),
        in_specs=[
            pl.BlockSpec((1, D2, SS), lambda ci, b: (b, ci, 0)),
            pl.BlockSpec((D2, SS), lambda ci, b: (ci, 0)),
        ],
        out_specs=pl.BlockSpec((1, D2, SS), lambda ci, b: (b, ci, 0)),
        out_shape=jax.ShapeDtypeStruct((B, D_MODEL, SS), x.dtype),
        compiler_params=pltpu.CompilerParams(
            dimension_semantics=("parallel", "parallel"),
        ),
    )(xf, pos)
    return out.reshape(x.shape)
